# Initial kernel scaffold; baseline (speedup 1.0000x reference)
import jax
import jax.numpy as jnp
from jax.experimental import pallas as pl


def kernel(z_flat, codebook, adjacency, current_sym):
    raise NotImplementedError("write your pallas kernel here")


# speedup vs baseline: 1.7190x; 1.7190x over previous
import jax
import jax.numpy as jnp
from jax import lax
from jax.experimental import pallas as pl
from jax.experimental.pallas import tpu as pltpu

_K = 8192
_D = 256
_RB = 8
_NSTEP = _K // 128 // _RB


def _body(sym_ref, z_ref, cb_ref, adj_ref, out_ref, dists_ref, rows_ref, sem):
    i = pl.program_id(0)

    @pl.when(i == 0)
    def _start_cur_row():
        pltpu.make_async_copy(
            adj_ref.at[pl.ds(sym_ref[0], 1)], rows_ref.at[pl.ds(0, 1)], sem
        ).start()

    @pl.when(i < _NSTEP)
    def _dist_step():
        z = z_ref[:].reshape(1, 1, _D)
        e = cb_ref[:] - z
        d = jnp.sum(e * e, axis=-1)
        dists_ref[pl.ds(i * _RB, _RB), :] = d

    @pl.when(i == _NSTEP)
    def _select():
        d = dists_ref[:]
        ri = lax.broadcasted_iota(jnp.int32, d.shape, 0)
        ci = lax.broadcasted_iota(jnp.int32, d.shape, 1)
        flat = ri * 128 + ci
        cur = sym_ref[0]

        idxs, vals = [], []
        for t in range(5):
            m = jnp.min(d)
            idx = jnp.min(jnp.where(d == m, flat, jnp.int32(1 << 30)))
            pltpu.make_async_copy(
                adj_ref.at[pl.ds(idx, 1)], rows_ref.at[pl.ds(t + 1, 1)], sem
            ).start()
            idxs.append(idx)
            vals.append(m)
            d = jnp.where(flat == idx, jnp.float32(jnp.inf), d)

        for t in range(6):
            pltpu.make_async_copy(
                adj_ref.at[pl.ds(0, 1)], rows_ref.at[pl.ds(t, 1)], sem
            ).wait()

        cur_row = rows_ref[pl.ds(0, 1), :]
        best = jnp.int32(0)
        bs = jnp.float32(0)
        for t in range(5):
            gd = jnp.mean(jnp.abs(cur_row - rows_ref[pl.ds(t + 1, 1), :]))
            sc = -vals[t] + jnp.float32(0.1) * gd
            sc = jnp.where(idxs[t] == cur, -jnp.inf, sc)
            if t == 0:
                best, bs = idxs[t], sc
            else:
                take = sc > bs
                best = jnp.where(take, idxs[t], best)
                bs = jnp.maximum(bs, sc)
        out_ref[0] = best


def kernel(z_flat, codebook, adjacency, current_sym):
    sym = jnp.asarray(current_sym, dtype=jnp.int32).reshape(1)
    z2 = z_flat.reshape(1, _D)
    cb3 = codebook.reshape(_K // 128, 128, _D)
    out = pl.pallas_call(
        _body,
        grid=(_NSTEP + 1,),
        in_specs=[
            pl.BlockSpec(memory_space=pltpu.SMEM),
            pl.BlockSpec((1, _D), lambda i: (0, 0)),
            pl.BlockSpec((_RB, 128, _D), lambda i: (jnp.minimum(i, _NSTEP - 1), 0, 0)),
            pl.BlockSpec(memory_space=pl.ANY),
        ],
        out_specs=pl.BlockSpec(memory_space=pltpu.SMEM),
        out_shape=jax.ShapeDtypeStruct((1,), jnp.int32),
        scratch_shapes=[
            pltpu.VMEM((_K // 128, 128), jnp.float32),
            pltpu.VMEM((8, _K), jnp.float32),
            pltpu.SemaphoreType.DMA,
        ],
    )(sym, z2, cb3, adjacency)
    return out[0]
